# baseline (speedup 1.0000x reference)
import functools

import jax
import jax.numpy as jnp
from jax import lax
from jax.experimental import pallas as pl
from jax.experimental.pallas import tpu as pltpu
from jax.experimental.pallas import tpu_sc as plsc

_T, _N, _E, _D, _H = 8, 10000, 320000, 128, 32
_NC, _NS = 2, 16
_NW = _NC * _NS
_EPT = _E // _NW
_CH = 125
_NCH = _EPT // _CH
_NPT = _N // _NS
_NBA = 5
_BNA = _N // _NBA
_NB = 10
_BN = _N // _NB

_mesh = plsc.VectorSubcoreMesh(core_axis_name="c", subcore_axis_name="s")


_NP = 10240
_ECH = _E // 128
_BMAX = 79


@functools.partial(
    pl.kernel,
    mesh=_mesh,
    out_type=jax.ShapeDtypeStruct((_NC, _T, _NP), jnp.float32),
    scratch_types=[
        pltpu.VMEM((_BMAX, 2, 128), jnp.int32),
        pltpu.VMEM((128,), jnp.float32),
        pltpu.VMEM_SHARED((_NP,), jnp.float32),
        pltpu.SemaphoreType.DMA,
        pltpu.SemaphoreType.DMA,
    ],
)
def _sc_degree(ei_hbm, zer_hbm, one_hbm, out_hbm, buf_v, ones_v, deg_sh,
               gsem, ssem):
    c = lax.axis_index("c")
    s = lax.axis_index("s")
    w = c * _NS + s
    c0 = (w * _ECH) // _NW
    c1 = ((w + 1) * _ECH) // _NW
    nch = c1 - c0
    sl = pl.ds(s * (_NP // _NS), _NP // _NS)
    pltpu.sync_copy(one_hbm, ones_v)
    for t in range(_T):
        pltpu.sync_copy(zer_hbm, deg_sh.at[sl])

        def _load(j, carry):
            pltpu.async_copy(ei_hbm.at[t, :, pl.ds((c0 + j) * 128, 128)],
                             buf_v.at[j], gsem)
            return carry

        lax.fori_loop(0, nch, _load, 0)
        plsc.subcore_barrier()

        def _chunk(j, carry):
            pltpu.make_async_copy(ei_hbm.at[t, :, pl.ds((c0 + j) * 128, 128)],
                                  buf_v.at[j], gsem).wait()
            pltpu.async_copy(ones_v, deg_sh.at[buf_v.at[j, 1]], ssem,
                             add=True)
            return carry

        lax.fori_loop(0, nch, _chunk, 0)

        def _drain(j, carry):
            pltpu.make_async_copy(ones_v, deg_sh.at[buf_v.at[j, 1]],
                                  ssem).wait()
            return carry

        lax.fori_loop(0, nch, _drain, 0)
        plsc.subcore_barrier()
        pltpu.sync_copy(deg_sh.at[sl], out_hbm.at[c, t, sl])
        plsc.subcore_barrier()


@functools.partial(
    pl.kernel,
    mesh=_mesh,
    compiler_params=pltpu.CompilerParams(use_tc_tiling_on_sc=False),
    out_type=jax.ShapeDtypeStruct((_NC, _N, _H), jnp.float32),
    scratch_types=[
        pltpu.VMEM((_NCH, _CH), jnp.int32),
        pltpu.VMEM((_NCH, _CH), jnp.int32),
        pltpu.VMEM((8, _CH, _H), jnp.float32),
        pltpu.VMEM_SHARED((_N, _H), jnp.float32),
        pltpu.SemaphoreType.DMA,
        pltpu.SemaphoreType.DMA,
        pltpu.SemaphoreType.DMA,
        pltpu.SemaphoreType.DMA,
        pltpu.SemaphoreType.DMA,
        pltpu.SemaphoreType.DMA,
        pltpu.SemaphoreType.DMA,
        pltpu.SemaphoreType.DMA,
    ],
)
def _sc_aggregate_t(ei_hbm, hp_hbm, z2_hbm, out_hbm,
                    src_v, dst_v, rows_v, agg_sh,
                    sem0, sem1, sem2, sem3, sem4, sem5, sem6, sem7):
    sems = (sem0, sem1, sem2, sem3, sem4, sem5, sem6, sem7)
    nbuf = len(sems)
    c = lax.axis_index("c")
    s = lax.axis_index("s")
    wid = c * _NS + s
    pltpu.sync_copy(ei_hbm.at[0, wid], src_v)
    pltpu.sync_copy(ei_hbm.at[1, wid], dst_v)

    @pl.when(s < _NS - 1)
    def _():
        pltpu.sync_copy(z2_hbm, agg_sh.at[pl.ds(s * 640, 640)])

    @pl.when(s == _NS - 1)
    def _():
        pltpu.sync_copy(z2_hbm.at[pl.ds(0, 400)], agg_sh.at[pl.ds(9600, 400)])

    plsc.subcore_barrier()

    for b in range(nbuf):
        pltpu.async_copy(hp_hbm.at[src_v.at[b]], rows_v.at[b], sems[b])

    def _wave(i, carry):
        for b in range(nbuf):
            j = i * nbuf + b
            pltpu.make_async_copy(hp_hbm.at[src_v.at[j]],
                                  rows_v.at[b], sems[b]).wait()
            pltpu.sync_copy(rows_v.at[b], agg_sh.at[dst_v.at[j]], add=True)

            @pl.when(j + nbuf < _NCH)
            def _():
                pltpu.async_copy(hp_hbm.at[src_v.at[j + nbuf]],
                                 rows_v.at[b], sems[b])
        return carry

    lax.fori_loop(0, _NCH // nbuf, _wave, 0)
    plsc.subcore_barrier()

    @pl.when(s < _NS - 1)
    def _():
        sl = pl.ds(s * 640, 640)
        pltpu.sync_copy(agg_sh.at[sl], out_hbm.at[c, sl])

    @pl.when(s == _NS - 1)
    def _():
        sl = pl.ds(9600, 400)
        pltpu.sync_copy(agg_sh.at[sl], out_hbm.at[c, sl])


def _tc_hprime_body(x_ref, w_ref, deg_ref, hp_ref, dis_ref):
    xb = x_ref[0, 0]
    deg = deg_ref[0, 0, 0, 0] + deg_ref[1, 0, 0, 0] + 1.0
    dis = lax.rsqrt(deg)
    h = jnp.dot(xb, w_ref[...], preferred_element_type=jnp.float32)
    hp_ref[0, 0] = h * dis[:, None]
    dis_ref[0, 0, 0] = dis


def _tc_hprime(x4, W, deg5):
    return pl.pallas_call(
        _tc_hprime_body,
        grid=(_T, _NBA),
        in_specs=[
            pl.BlockSpec((1, 1, _BNA, _D), lambda t, n: (t, n, 0, 0)),
            pl.BlockSpec((_D, _H), lambda t, n: (0, 0)),
            pl.BlockSpec((_NC, 1, 1, 1, _BNA), lambda t, n: (0, t, n, 0, 0)),
        ],
        out_specs=[
            pl.BlockSpec((1, 1, _BNA, _H), lambda t, n: (t, n, 0, 0)),
            pl.BlockSpec((1, 1, 1, _BNA), lambda t, n: (t, n, 0, 0)),
        ],
        out_shape=[
            jax.ShapeDtypeStruct((_T, _NBA, _BNA, _H), jnp.float32),
            jax.ShapeDtypeStruct((_T, _NBA, 1, _BNA), jnp.float32),
        ],
    )(x4, W, deg5)


def _tc_step_body(agg_ref, hp_ref, dis_ref, h_ref, c_ref, b_ref, wih_ref,
                  whh_ref, bih_ref, bhh_ref, ho_ref, co_ref):
    a = agg_ref[0, 0] + agg_ref[1, 0] + hp_ref[0, 0]
    dis = dis_ref[0, 0, 0]
    g = jnp.maximum(a * dis[:, None] + b_ref[0], 0.0)
    h = h_ref[0]
    c = c_ref[0]
    gates = []
    for k in range(4):
        gk = (jnp.dot(g, wih_ref[k], preferred_element_type=jnp.float32)
              + jnp.dot(h, whh_ref[k], preferred_element_type=jnp.float32)
              + bih_ref[k][None, :] + bhh_ref[k][None, :])
        gates.append(gk)
    i_g = jax.nn.sigmoid(gates[0])
    f_g = jax.nn.sigmoid(gates[1])
    g_g = jnp.tanh(gates[2])
    o_g = jax.nn.sigmoid(gates[3])
    cn = f_g * c + i_g * g_g
    co_ref[0] = cn
    ho_ref[0] = o_g * jnp.tanh(cn)


def _tc_step(t, agg4, hp, dis, h, c, b2, wihs, whhs, bih2, bhh2):
    return pl.pallas_call(
        _tc_step_body,
        grid=(_NBA,),
        in_specs=[
            pl.BlockSpec((_NC, 1, _BNA, _H), lambda n: (0, n, 0, 0)),
            pl.BlockSpec((1, 1, _BNA, _H), lambda n, _t=t: (_t, n, 0, 0)),
            pl.BlockSpec((1, 1, 1, _BNA), lambda n, _t=t: (_t, n, 0, 0)),
            pl.BlockSpec((1, _BNA, _H), lambda n: (n, 0, 0)),
            pl.BlockSpec((1, _BNA, _H), lambda n: (n, 0, 0)),
            pl.BlockSpec((1, _H), lambda n: (0, 0)),
            pl.BlockSpec((4, _H, _H), lambda n: (0, 0, 0)),
            pl.BlockSpec((4, _H, _H), lambda n: (0, 0, 0)),
            pl.BlockSpec((4, _H), lambda n: (0, 0)),
            pl.BlockSpec((4, _H), lambda n: (0, 0)),
        ],
        out_specs=[
            pl.BlockSpec((1, _BNA, _H), lambda n: (n, 0, 0)),
            pl.BlockSpec((1, _BNA, _H), lambda n: (n, 0, 0)),
        ],
        out_shape=[
            jax.ShapeDtypeStruct((_NBA, _BNA, _H), jnp.float32),
            jax.ShapeDtypeStruct((_NBA, _BNA, _H), jnp.float32),
        ],
    )(agg4, hp, dis, h, c, b2, wihs, whhs, bih2, bhh2)


def kernel(x, edge_index, W, b, W_ih, W_hh, b_ih, b_hh):
    x4 = x.reshape(_T, _NBA, _BNA, _D)
    toff = (jnp.arange(_T, dtype=edge_index.dtype) * _N)[:, None]
    ei_off = jnp.stack([edge_index[:, 0] + toff, edge_index[:, 1]], axis=1)
    ei = ei_off.reshape(_T, 2, _NW, _NCH, _CH)
    zer = jnp.zeros((640,), jnp.float32)
    one = jnp.ones((128,), jnp.float32)
    z2 = jnp.zeros((640, _H), jnp.float32)
    wihs = W_ih.reshape(4, _H, _H).transpose(0, 2, 1)
    whhs = W_hh.reshape(4, _H, _H).transpose(0, 2, 1)
    bih2 = b_ih.reshape(4, _H)
    bhh2 = b_hh.reshape(4, _H)
    b2 = b.reshape(1, _H)

    deg = _sc_degree(edge_index, zer, one)
    deg5 = deg[:, :, :_N].reshape(_NC, _T, _NBA, 1, _BNA)
    hp, dis = _tc_hprime(x4, W, deg5)
    hp_flat = hp.reshape(_T * _N, _H)
    h = jnp.zeros((_NBA, _BNA, _H), jnp.float32)
    c = jnp.zeros((_NBA, _BNA, _H), jnp.float32)
    for t in range(_T):
        agg_t = _sc_aggregate_t(ei[t], hp_flat, z2)
        agg4 = agg_t.reshape(_NC, _NBA, _BNA, _H)
        h, c = _tc_step(t, agg4, hp, dis, h, c, b2, wihs, whhs, bih2, bhh2)
    return h.reshape(_N, _H)


# speedup vs baseline: 1.0708x; 1.0708x over previous
import functools

import jax
import jax.numpy as jnp
from jax import lax
from jax.experimental import pallas as pl
from jax.experimental.pallas import tpu as pltpu
from jax.experimental.pallas import tpu_sc as plsc

_T, _N, _E, _D, _H = 8, 10000, 320000, 128, 32
_NC, _NS = 2, 16
_NW = _NC * _NS
_EPT = _E // _NW
_CH = 125
_NCH = _EPT // _CH
_NPT = _N // _NS
_NBA = 5
_BNA = _N // _NBA
_NB = 10
_BN = _N // _NB

_mesh = plsc.VectorSubcoreMesh(core_axis_name="c", subcore_axis_name="s")


_NP = 10240
_ECH = _E // 128
_BMAX = 79


@functools.partial(
    pl.kernel,
    mesh=_mesh,
    out_type=jax.ShapeDtypeStruct((_NC, _T, _NP), jnp.float32),
    scratch_types=[
        pltpu.VMEM((_BMAX, 2, 128), jnp.int32),
        pltpu.VMEM((128,), jnp.float32),
        pltpu.VMEM_SHARED((_NP,), jnp.float32),
        pltpu.SemaphoreType.DMA,
        pltpu.SemaphoreType.DMA,
    ],
)
def _sc_degree(ei_hbm, zer_hbm, one_hbm, out_hbm, buf_v, ones_v, deg_sh,
               gsem, ssem):
    c = lax.axis_index("c")
    s = lax.axis_index("s")
    w = c * _NS + s
    c0 = (w * _ECH) // _NW
    c1 = ((w + 1) * _ECH) // _NW
    nch = c1 - c0
    sl = pl.ds(s * (_NP // _NS), _NP // _NS)
    pltpu.sync_copy(one_hbm, ones_v)
    for t in range(_T):
        pltpu.sync_copy(zer_hbm, deg_sh.at[sl])

        def _load(j, carry):
            pltpu.async_copy(ei_hbm.at[t, :, pl.ds((c0 + j) * 128, 128)],
                             buf_v.at[j], gsem)
            return carry

        lax.fori_loop(0, nch, _load, 0)
        plsc.subcore_barrier()

        def _chunk(j, carry):
            pltpu.make_async_copy(ei_hbm.at[t, :, pl.ds((c0 + j) * 128, 128)],
                                  buf_v.at[j], gsem).wait()
            pltpu.async_copy(ones_v, deg_sh.at[buf_v.at[j, 1]], ssem,
                             add=True)
            return carry

        lax.fori_loop(0, nch, _chunk, 0)

        def _drain(j, carry):
            pltpu.make_async_copy(ones_v, deg_sh.at[buf_v.at[j, 1]],
                                  ssem).wait()
            return carry

        lax.fori_loop(0, nch, _drain, 0)
        plsc.subcore_barrier()
        pltpu.sync_copy(deg_sh.at[sl], out_hbm.at[c, t, sl])
        plsc.subcore_barrier()


@functools.partial(
    pl.kernel,
    mesh=_mesh,
    compiler_params=pltpu.CompilerParams(use_tc_tiling_on_sc=False),
    out_type=jax.ShapeDtypeStruct((_NC, _N, _H), jnp.float32),
    scratch_types=[
        pltpu.VMEM((_NCH, _CH), jnp.int32),
        pltpu.VMEM((_NCH, _CH), jnp.int32),
        pltpu.VMEM((8, _CH, _H), jnp.float32),
        pltpu.VMEM_SHARED((_N, _H), jnp.float32),
        pltpu.SemaphoreType.DMA,
        pltpu.SemaphoreType.DMA,
        pltpu.SemaphoreType.DMA,
        pltpu.SemaphoreType.DMA,
        pltpu.SemaphoreType.DMA,
        pltpu.SemaphoreType.DMA,
        pltpu.SemaphoreType.DMA,
        pltpu.SemaphoreType.DMA,
    ],
)
def _sc_aggregate_t(ei_hbm, hp_hbm, z2_hbm, out_hbm,
                    src_v, dst_v, rows_v, agg_sh,
                    sem0, sem1, sem2, sem3, sem4, sem5, sem6, sem7):
    sems = (sem0, sem1, sem2, sem3, sem4, sem5, sem6, sem7)
    nbuf = len(sems)
    c = lax.axis_index("c")
    s = lax.axis_index("s")
    wid = c * _NS + s
    pltpu.sync_copy(ei_hbm.at[0, wid], src_v)
    pltpu.sync_copy(ei_hbm.at[1, wid], dst_v)

    @pl.when(jnp.logical_and(c == 0, s < _NS - 1))
    def _():
        sl = pl.ds(s * 640, 640)
        pltpu.sync_copy(hp_hbm.at[sl], agg_sh.at[sl])

    @pl.when(jnp.logical_and(c == 0, s == _NS - 1))
    def _():
        sl = pl.ds(9600, 400)
        pltpu.sync_copy(hp_hbm.at[sl], agg_sh.at[sl])

    @pl.when(jnp.logical_and(c == 1, s < _NS - 1))
    def _():
        pltpu.sync_copy(z2_hbm, agg_sh.at[pl.ds(s * 640, 640)])

    @pl.when(jnp.logical_and(c == 1, s == _NS - 1))
    def _():
        pltpu.sync_copy(z2_hbm.at[pl.ds(0, 400)], agg_sh.at[pl.ds(9600, 400)])

    plsc.subcore_barrier()

    for b in range(nbuf):
        pltpu.async_copy(hp_hbm.at[src_v.at[b]], rows_v.at[b], sems[b])

    def _wave(i, carry):
        for b in range(nbuf):
            j = i * nbuf + b
            pltpu.make_async_copy(hp_hbm.at[src_v.at[j]],
                                  rows_v.at[b], sems[b]).wait()
            pltpu.sync_copy(rows_v.at[b], agg_sh.at[dst_v.at[j]], add=True)

            @pl.when(j + nbuf < _NCH)
            def _():
                pltpu.async_copy(hp_hbm.at[src_v.at[j + nbuf]],
                                 rows_v.at[b], sems[b])
        return carry

    lax.fori_loop(0, _NCH // nbuf, _wave, 0)
    plsc.subcore_barrier()

    @pl.when(s < _NS - 1)
    def _():
        sl = pl.ds(s * 640, 640)
        pltpu.sync_copy(agg_sh.at[sl], out_hbm.at[c, sl])

    @pl.when(s == _NS - 1)
    def _():
        sl = pl.ds(9600, 400)
        pltpu.sync_copy(agg_sh.at[sl], out_hbm.at[c, sl])


def _tc_hprime_body(x_ref, w_ref, deg_ref, hp_ref, dis_ref):
    xb = x_ref[0, 0]
    deg = deg_ref[0, 0, 0, 0] + deg_ref[1, 0, 0, 0] + 1.0
    dis = lax.rsqrt(deg)
    h = jnp.dot(xb, w_ref[...], preferred_element_type=jnp.float32)
    hp_ref[0, 0] = h * dis[:, None]
    dis_ref[0, 0, 0] = dis


def _tc_hprime(x4, W, deg5):
    return pl.pallas_call(
        _tc_hprime_body,
        grid=(_T, _NBA),
        in_specs=[
            pl.BlockSpec((1, 1, _BNA, _D), lambda t, n: (t, n, 0, 0)),
            pl.BlockSpec((_D, _H), lambda t, n: (0, 0)),
            pl.BlockSpec((_NC, 1, 1, 1, _BNA), lambda t, n: (0, t, n, 0, 0)),
        ],
        out_specs=[
            pl.BlockSpec((1, 1, _BNA, _H), lambda t, n: (t, n, 0, 0)),
            pl.BlockSpec((1, 1, 1, _BNA), lambda t, n: (t, n, 0, 0)),
        ],
        out_shape=[
            jax.ShapeDtypeStruct((_T, _NBA, _BNA, _H), jnp.float32),
            jax.ShapeDtypeStruct((_T, _NBA, 1, _BNA), jnp.float32),
        ],
    )(x4, W, deg5)


def _tc_step_body(agg_ref, dis_ref, h_ref, c_ref, b_ref, wih_ref,
                  whh_ref, bih_ref, bhh_ref, ho_ref, co_ref):
    a = agg_ref[0, 0] + agg_ref[1, 0]
    dis = dis_ref[0, 0, 0]
    g = jnp.maximum(a * dis[:, None] + b_ref[0], 0.0)
    h = h_ref[0]
    c = c_ref[0]
    gates = []
    for k in range(4):
        gk = (jnp.dot(g, wih_ref[k], preferred_element_type=jnp.float32)
              + jnp.dot(h, whh_ref[k], preferred_element_type=jnp.float32)
              + bih_ref[k][None, :] + bhh_ref[k][None, :])
        gates.append(gk)
    i_g = jax.nn.sigmoid(gates[0])
    f_g = jax.nn.sigmoid(gates[1])
    g_g = jnp.tanh(gates[2])
    o_g = jax.nn.sigmoid(gates[3])
    cn = f_g * c + i_g * g_g
    co_ref[0] = cn
    ho_ref[0] = o_g * jnp.tanh(cn)


def _tc_step(t, agg4, dis, h, c, b2, wihs, whhs, bih2, bhh2):
    return pl.pallas_call(
        _tc_step_body,
        grid=(_NBA,),
        in_specs=[
            pl.BlockSpec((_NC, 1, _BNA, _H), lambda n: (0, n, 0, 0)),
            pl.BlockSpec((1, 1, 1, _BNA), lambda n, _t=t: (_t, n, 0, 0)),
            pl.BlockSpec((1, _BNA, _H), lambda n: (n, 0, 0)),
            pl.BlockSpec((1, _BNA, _H), lambda n: (n, 0, 0)),
            pl.BlockSpec((1, _H), lambda n: (0, 0)),
            pl.BlockSpec((4, _H, _H), lambda n: (0, 0, 0)),
            pl.BlockSpec((4, _H, _H), lambda n: (0, 0, 0)),
            pl.BlockSpec((4, _H), lambda n: (0, 0)),
            pl.BlockSpec((4, _H), lambda n: (0, 0)),
        ],
        out_specs=[
            pl.BlockSpec((1, _BNA, _H), lambda n: (n, 0, 0)),
            pl.BlockSpec((1, _BNA, _H), lambda n: (n, 0, 0)),
        ],
        out_shape=[
            jax.ShapeDtypeStruct((_NBA, _BNA, _H), jnp.float32),
            jax.ShapeDtypeStruct((_NBA, _BNA, _H), jnp.float32),
        ],
    )(agg4, dis, h, c, b2, wihs, whhs, bih2, bhh2)


def kernel(x, edge_index, W, b, W_ih, W_hh, b_ih, b_hh):
    x4 = x.reshape(_T, _NBA, _BNA, _D)
    ei = edge_index.reshape(_T, 2, _NW, _NCH, _CH)
    zer = jnp.zeros((640,), jnp.float32)
    one = jnp.ones((128,), jnp.float32)
    z2 = jnp.zeros((640, _H), jnp.float32)
    wihs = W_ih.reshape(4, _H, _H).transpose(0, 2, 1)
    whhs = W_hh.reshape(4, _H, _H).transpose(0, 2, 1)
    bih2 = b_ih.reshape(4, _H)
    bhh2 = b_hh.reshape(4, _H)
    b2 = b.reshape(1, _H)

    deg = _sc_degree(edge_index, zer, one)
    deg5 = deg[:, :, :_N].reshape(_NC, _T, _NBA, 1, _BNA)
    hp, dis = _tc_hprime(x4, W, deg5)
    hp2 = hp.reshape(_T, _N, _H)
    h = jnp.zeros((_NBA, _BNA, _H), jnp.float32)
    c = jnp.zeros((_NBA, _BNA, _H), jnp.float32)
    for t in range(_T):
        agg_t = _sc_aggregate_t(ei[t], hp2[t], z2)
        agg4 = agg_t.reshape(_NC, _NBA, _BNA, _H)
        h, c = _tc_step(t, agg4, dis, h, c, b2, wihs, whhs, bih2, bhh2)
    return h.reshape(_N, _H)
